# baseline (speedup 1.0000x reference)
import functools

import jax
import jax.numpy as jnp
from jax import lax
from jax.experimental import pallas as pl
from jax.experimental.pallas import tpu as pltpu
from jax.experimental.pallas import tpu_sc as plsc

EMBED_DIM = 64
SPLIT = (104, 96)


@functools.lru_cache(maxsize=None)
def _build(batch, hist):
    info = plsc.get_sparse_core_info()
    nc, ns = info.num_cores, info.num_subcores
    nw = nc * ns
    rows_per_w = batch // nw
    assert rows_per_w * nw == batch and sum(SPLIT) == hist

    mesh = plsc.VectorSubcoreMesh(core_axis_name="c", subcore_axis_name="s")

    @functools.partial(
        pl.kernel,
        out_type=jax.ShapeDtypeStruct((batch, hist, EMBED_DIM), jnp.float32),
        mesh=mesh,
        scratch_types=[
            pltpu.VMEM((rows_per_w, hist), jnp.int32),
            [pltpu.VMEM((hist, EMBED_DIM), jnp.float32) for _ in range(2)],
            [pltpu.SemaphoreType.DMA for _ in range(2)],
            [pltpu.SemaphoreType.DMA for _ in range(2)],
        ],
        compiler_params=pltpu.CompilerParams(use_tc_tiling_on_sc=False),
    )
    def gather_kernel(idx_hbm, table_hbm, out_hbm, idx_v, rows, gsem, wsem):
        wid = lax.axis_index("s") * nc + lax.axis_index("c")
        base = wid * rows_per_w

        pltpu.sync_copy(idx_hbm.at[pl.ds(base, rows_per_w)], idx_v)

        def fire(r, b):
            off = 0
            for n in SPLIT:
                pltpu.async_copy(
                    table_hbm.at[idx_v.at[r, pl.ds(off, n)]],
                    rows[b].at[pl.ds(off, n)],
                    gsem[b])
                off += n

        def drain_gathers(r, b):
            off = 0
            for n in SPLIT:
                pltpu.make_async_copy(
                    table_hbm.at[idx_v.at[r, pl.ds(off, n)]],
                    rows[b].at[pl.ds(off, n)],
                    gsem[b]).wait()
                off += n

        def start_write(r, b):
            pltpu.async_copy(rows[b], out_hbm.at[base + r], wsem[b])

        def wait_write(r, b):
            pltpu.make_async_copy(rows[b], out_hbm.at[base + r], wsem[b]).wait()

        fire(0, 0)

        @pl.loop(0, rows_per_w, step=2)
        def _(r0):
            for b in range(2):
                r = r0 + b
                drain_gathers(r, b)
                nb = 1 - b

                @pl.when(r + 1 < rows_per_w)
                def _():
                    @pl.when(r + 1 >= 2)
                    def _():
                        wait_write(r - 1, nb)
                    fire(r + 1, nb)

                start_write(r, b)

        for b in range(2):
            wait_write(rows_per_w - 2 + b, b)

    return gather_kernel


def kernel(indices, embeddings):
    batch, hist = indices.shape
    run = _build(batch, hist)
    return run(indices, embeddings)
